# baseline (speedup 1.0000x reference)
import jax
import jax.numpy as jnp
from jax.experimental import pallas as pl
from jax.experimental.pallas import tpu as pltpu

_STRIP = 1024


def _linear_kernel(x_ref, w_ref, b_ref, o_hbm, acc_ref, sems):
    j = pl.program_id(0)
    k = pl.program_id(1)
    nj = pl.num_programs(0)
    nk = pl.num_programs(1)
    _, tm, n = acc_ref.shape
    slot = jax.lax.rem(j, 2)
    acc = acc_ref.at[slot]

    def writeback(jj, sl):
        return pltpu.make_async_copy(
            acc_ref.at[sl], o_hbm.at[pl.ds(jj * tm, tm), :], sems.at[sl]
        )

    if nj > 2:
        @pl.when((k == 0) & (j >= 2))
        def _():
            writeback(j - 2, slot).wait()

    xb = x_ref[...].astype(jnp.bfloat16)
    for s in range(0, n, _STRIP):
        sl = pl.ds(s, min(_STRIP, n - s))
        prod = jnp.dot(
            xb,
            w_ref[:, sl].astype(jnp.bfloat16),
            preferred_element_type=jnp.float32,
        )

        @pl.when(k == 0)
        def _(prod=prod, sl=sl):
            acc[:, sl] = prod + b_ref[:, sl]

        @pl.when(k != 0)
        def _(prod=prod, sl=sl):
            acc[:, sl] += prod

    @pl.when(k == nk - 1)
    def _():
        writeback(j, slot).start()

    if nj >= 2:
        @pl.when((j == nj - 1) & (k == nk - 1))
        def _():
            writeback(nj - 2, jax.lax.rem(nj - 2, 2)).wait()
            writeback(nj - 1, jax.lax.rem(nj - 1, 2)).wait()
    else:
        @pl.when((j == nj - 1) & (k == nk - 1))
        def _():
            writeback(0, 0).wait()


def kernel(x, wt, bias):
    k_pad, n_pad = wt.shape
    orig_lead = x.shape[:-1]
    feat = x.shape[-1]
    assert feat == k_pad, "activations must match the prepadded weight K"

    x2d = x.reshape(-1, feat)
    m = x2d.shape[0]
    b32 = bias.astype(jnp.float32)

    tm = m // 4 if m % 4 == 0 and (m // 4) % 8 == 0 else m
    tk = 512
    while k_pad % tk:
        tk //= 2
    grid = (m // tm, k_pad // tk)

    cost = pl.CostEstimate(
        flops=2 * m * k_pad * n_pad,
        transcendentals=0,
        bytes_accessed=(m * k_pad + k_pad * n_pad + n_pad + m * n_pad) * 4,
    )

    y2d = pl.pallas_call(
        _linear_kernel,
        out_shape=jax.ShapeDtypeStruct((m, n_pad), jnp.float32),
        grid_spec=pltpu.PrefetchScalarGridSpec(
            num_scalar_prefetch=0,
            grid=grid,
            in_specs=[
                pl.BlockSpec((tm, tk), lambda j, k: (j, k)),
                pl.BlockSpec((tk, n_pad), lambda j, k: (k, 0)),
                pl.BlockSpec((1, n_pad), lambda j, k: (0, 0)),
            ],
            out_specs=pl.BlockSpec(memory_space=pl.ANY),
            scratch_shapes=[
                pltpu.VMEM((2, tm, n_pad), jnp.float32),
                pltpu.SemaphoreType.DMA((2,)),
            ],
        ),
        compiler_params=pltpu.CompilerParams(
            dimension_semantics=("arbitrary", "arbitrary"),
            vmem_limit_bytes=60 * 1024 * 1024,
        ),
        cost_estimate=cost,
    )(x2d, wt, b32)

    return y2d.reshape(*orig_lead, n_pad)


# speedup vs baseline: 1.3350x; 1.3350x over previous
import jax
import jax.numpy as jnp
from jax.experimental import pallas as pl
from jax.experimental.pallas import tpu as pltpu

_STRIP = 1024


def _linear_kernel(x_ref, w_ref, b_ref, o_hbm, acc_ref, sems):
    j = pl.program_id(0)
    k = pl.program_id(1)
    nj = pl.num_programs(0)
    nk = pl.num_programs(1)
    _, tm, n = acc_ref.shape
    slot = jax.lax.rem(j, 2)
    acc = acc_ref.at[slot]

    def writeback(jj, sl):
        return pltpu.make_async_copy(
            acc_ref.at[sl], o_hbm.at[pl.ds(jj * tm, tm), :], sems.at[sl]
        )

    if nj > 2:
        @pl.when((k == 0) & (j >= 2))
        def _():
            writeback(j - 2, slot).wait()

    @pl.when(k == 0)
    def _():
        acc[...] = jnp.broadcast_to(b_ref[...], acc.shape)

    xb = x_ref[...].astype(jnp.bfloat16)
    for s in range(0, n, _STRIP):
        sl = pl.ds(s, min(_STRIP, n - s))
        acc[:, sl] += jnp.dot(
            xb,
            w_ref[:, sl].astype(jnp.bfloat16),
            preferred_element_type=jnp.float32,
        )

    @pl.when(k == nk - 1)
    def _():
        writeback(j, slot).start()

    if nj >= 2:
        @pl.when((j == nj - 1) & (k == nk - 1))
        def _():
            writeback(nj - 2, jax.lax.rem(nj - 2, 2)).wait()
            writeback(nj - 1, jax.lax.rem(nj - 1, 2)).wait()
    else:
        @pl.when((j == nj - 1) & (k == nk - 1))
        def _():
            writeback(0, 0).wait()


def kernel(x, wt, bias):
    k_pad, n_pad = wt.shape
    orig_lead = x.shape[:-1]
    feat = x.shape[-1]
    assert feat == k_pad, "activations must match the prepadded weight K"

    x2d = x.reshape(-1, feat)
    m = x2d.shape[0]
    b32 = bias.astype(jnp.float32)

    tm = m // 4 if m % 4 == 0 and (m // 4) % 8 == 0 else m
    tk = 512
    while k_pad % tk:
        tk //= 2
    grid = (m // tm, k_pad // tk)

    cost = pl.CostEstimate(
        flops=2 * m * k_pad * n_pad,
        transcendentals=0,
        bytes_accessed=(m * k_pad + k_pad * n_pad + n_pad + m * n_pad) * 4,
    )

    y2d = pl.pallas_call(
        _linear_kernel,
        out_shape=jax.ShapeDtypeStruct((m, n_pad), jnp.float32),
        grid_spec=pltpu.PrefetchScalarGridSpec(
            num_scalar_prefetch=0,
            grid=grid,
            in_specs=[
                pl.BlockSpec((tm, tk), lambda j, k: (j, k)),
                pl.BlockSpec((tk, n_pad), lambda j, k: (k, 0)),
                pl.BlockSpec((1, n_pad), lambda j, k: (0, 0)),
            ],
            out_specs=pl.BlockSpec(memory_space=pl.ANY),
            scratch_shapes=[
                pltpu.VMEM((2, tm, n_pad), jnp.float32),
                pltpu.SemaphoreType.DMA((2,)),
            ],
        ),
        compiler_params=pltpu.CompilerParams(
            dimension_semantics=("arbitrary", "arbitrary"),
            vmem_limit_bytes=60 * 1024 * 1024,
        ),
        cost_estimate=cost,
    )(x2d, wt, b32)

    return y2d.reshape(*orig_lead, n_pad)


# speedup vs baseline: 1.3739x; 1.0291x over previous
import jax
import jax.numpy as jnp
from jax.experimental import pallas as pl
from jax.experimental.pallas import tpu as pltpu

_STRIP = 1024


def _linear_kernel(x_ref, w_ref, b_ref, o_hbm, acc_ref, sems):
    j = pl.program_id(0)
    k = pl.program_id(1)
    nj = pl.num_programs(0)
    nk = pl.num_programs(1)
    _, tm, n = acc_ref.shape
    slot = jax.lax.rem(j, 2)
    acc = acc_ref.at[slot]

    def writeback(jj, sl):
        return pltpu.make_async_copy(
            acc_ref.at[sl], o_hbm.at[pl.ds(jj * tm, tm), :], sems.at[sl]
        )

    if nj > 2:
        @pl.when((k == 0) & (j >= 2))
        def _():
            writeback(j - 2, slot).wait()

    xb = x_ref[...].astype(jnp.bfloat16)
    first = k == 0
    for s in range(0, n, _STRIP):
        sl = pl.ds(s, min(_STRIP, n - s))
        prod = jnp.dot(
            xb,
            w_ref[:, sl].astype(jnp.bfloat16),
            preferred_element_type=jnp.float32,
        )
        base = jnp.where(first, jnp.broadcast_to(b_ref[:, sl], prod.shape),
                         acc[:, sl])
        acc[:, sl] = base + prod

    @pl.when(k == nk - 1)
    def _():
        writeback(j, slot).start()

    if nj >= 2:
        @pl.when((j == nj - 1) & (k == nk - 1))
        def _():
            writeback(nj - 2, jax.lax.rem(nj - 2, 2)).wait()
            writeback(nj - 1, jax.lax.rem(nj - 1, 2)).wait()
    else:
        @pl.when((j == nj - 1) & (k == nk - 1))
        def _():
            writeback(0, 0).wait()


def kernel(x, wt, bias):
    k_pad, n_pad = wt.shape
    orig_lead = x.shape[:-1]
    feat = x.shape[-1]
    assert feat == k_pad, "activations must match the prepadded weight K"

    x2d = x.reshape(-1, feat)
    m = x2d.shape[0]
    b32 = bias.astype(jnp.float32)

    tm = m // 4 if m % 4 == 0 and (m // 4) % 8 == 0 else m
    tk = 512
    while k_pad % tk:
        tk //= 2
    grid = (m // tm, k_pad // tk)

    cost = pl.CostEstimate(
        flops=2 * m * k_pad * n_pad,
        transcendentals=0,
        bytes_accessed=(m * k_pad + k_pad * n_pad + n_pad + m * n_pad) * 4,
    )

    y2d = pl.pallas_call(
        _linear_kernel,
        out_shape=jax.ShapeDtypeStruct((m, n_pad), jnp.float32),
        grid_spec=pltpu.PrefetchScalarGridSpec(
            num_scalar_prefetch=0,
            grid=grid,
            in_specs=[
                pl.BlockSpec((tm, tk), lambda j, k: (j, k)),
                pl.BlockSpec((tk, n_pad), lambda j, k: (k, 0)),
                pl.BlockSpec((1, n_pad), lambda j, k: (0, 0)),
            ],
            out_specs=pl.BlockSpec(memory_space=pl.ANY),
            scratch_shapes=[
                pltpu.VMEM((2, tm, n_pad), jnp.float32),
                pltpu.SemaphoreType.DMA((2,)),
            ],
        ),
        compiler_params=pltpu.CompilerParams(
            dimension_semantics=("arbitrary", "arbitrary"),
            vmem_limit_bytes=60 * 1024 * 1024,
        ),
        cost_estimate=cost,
    )(x2d, wt, b32)

    return y2d.reshape(*orig_lead, n_pad)
